# baseline (device time: 506405 ns/iter reference)
import jax
import jax.numpy as jnp
from jax import lax
from jax.experimental import pallas as pl
from jax.experimental.pallas import tpu as pltpu

N_DEV = 16


def kernel(x, w_mat):
    m, k_sh = x.shape
    _, n = w_mat.shape
    chunk = m // N_DEV

    def body(x_ref, w_ref, out_ref, comm_ref,
             rs_send_sems, rs_recv_sems, ag_send_sems, ag_recv_sems,
             hop_sem):
        p = lax.axis_index("i")
        left = (p - 1) % N_DEV
        right = (p + 1) % N_DEV

        barrier_sem = pltpu.get_barrier_semaphore()
        for nbr in (left, right):
            pl.semaphore_signal(barrier_sem, inc=1, device_id=(nbr,),
                                device_id_type=pl.DeviceIdType.MESH)
        pl.semaphore_wait(barrier_sem, 2)

        def hop_barrier():
            for nbr in (left, right):
                pl.semaphore_signal(hop_sem, inc=1, device_id=(nbr,),
                                    device_id_type=pl.DeviceIdType.MESH)
            pl.semaphore_wait(hop_sem, 2)

        out_ref[:, :] = jnp.dot(x_ref[:, :], w_ref[:, :],
                                preferred_element_type=jnp.float32)

        for h in range(N_DEV - 1):
            slot = h % 2
            c_send = (p - h) % N_DEV
            c_recv = (p - h - 1) % N_DEV
            rdma = pltpu.make_async_remote_copy(
                src_ref=out_ref.at[pl.ds(c_send * chunk, chunk), :],
                dst_ref=comm_ref.at[slot],
                send_sem=rs_send_sems.at[slot],
                recv_sem=rs_recv_sems.at[slot],
                device_id=(right,),
                device_id_type=pl.DeviceIdType.MESH,
            )
            rdma.start()
            rdma.wait()
            rows = pl.ds(c_recv * chunk, chunk)
            out_ref[rows, :] = out_ref[rows, :] + comm_ref[slot]
            hop_barrier()

        c_own = (p + 1) % N_DEV
        rows = pl.ds(c_own * chunk, chunk)
        y = out_ref[rows, :]
        c = 0.7978845608028654
        out_ref[rows, :] = 0.5 * y * (1.0 + jnp.tanh(c * (y + 0.044715 * y * y * y)))

        for t in range(N_DEV - 1):
            slot = t % 2
            c_s = (p + 1 - t) % N_DEV
            rows_s = pl.ds(c_s * chunk, chunk)
            rdma = pltpu.make_async_remote_copy(
                src_ref=out_ref.at[rows_s, :],
                dst_ref=out_ref.at[rows_s, :],
                send_sem=ag_send_sems.at[slot],
                recv_sem=ag_recv_sems.at[slot],
                device_id=(right,),
                device_id_type=pl.DeviceIdType.MESH,
            )
            rdma.start()
            rdma.wait()
            hop_barrier()

    return pl.pallas_call(
        body,
        out_shape=jax.ShapeDtypeStruct((m, n), jnp.float32),
        in_specs=[
            pl.BlockSpec(memory_space=pltpu.VMEM),
            pl.BlockSpec(memory_space=pltpu.VMEM),
        ],
        out_specs=pl.BlockSpec(memory_space=pltpu.VMEM),
        scratch_shapes=[
            pltpu.VMEM((2, chunk, n), jnp.float32),
            pltpu.SemaphoreType.DMA((2,)),
            pltpu.SemaphoreType.DMA((2,)),
            pltpu.SemaphoreType.DMA((2,)),
            pltpu.SemaphoreType.DMA((2,)),
            pltpu.SemaphoreType.REGULAR,
        ],
        compiler_params=pltpu.CompilerParams(collective_id=0),
    )(x, w_mat)


# device time: 202967 ns/iter; 2.4950x vs baseline; 2.4950x over previous
import jax
import jax.numpy as jnp
from jax import lax
from jax.experimental import pallas as pl
from jax.experimental.pallas import tpu as pltpu

N_DEV = 16
S = 4


def kernel(x, w_mat):
    m, k_sh = x.shape
    _, n = w_mat.shape
    chunk = m // N_DEV
    half = n // 2
    sub = half // S

    def body(x_ref, w_ref, out_ref, comm_ref,
             rs_send, rs_recv, ag_send, ag_recv):
        p = lax.axis_index("i")
        left = (p - 1) % N_DEV
        right = (p + 1) % N_DEV

        barrier_sem = pltpu.get_barrier_semaphore()
        for nbr in (left, right):
            pl.semaphore_signal(barrier_sem, inc=1, device_id=(nbr,),
                                device_id_type=pl.DeviceIdType.MESH)
        pl.semaphore_wait(barrier_sem, 2)

        out_ref[:, :] = jnp.dot(x_ref[:, :], w_ref[:, :],
                                preferred_element_type=jnp.float32)

        def rs_copy(h, d, s_i):
            c_send = (p - h) % N_DEV if d == 0 else (p + h) % N_DEV
            tgt = right if d == 0 else left
            co = d * half + s_i * sub
            return pltpu.make_async_remote_copy(
                src_ref=out_ref.at[pl.ds(c_send * chunk, chunk),
                                   pl.ds(co, sub)],
                dst_ref=comm_ref.at[h, d, s_i],
                send_sem=rs_send.at[h, d, s_i],
                recv_sem=rs_recv.at[h, d, s_i],
                device_id=(tgt,),
                device_id_type=pl.DeviceIdType.MESH,
            )

        for d in range(2):
            for s_i in range(S):
                rs_copy(0, d, s_i).start()
        for h in range(N_DEV - 1):
            for d in range(2):
                c_recv = (p - h - 1) % N_DEV if d == 0 else (p + h + 1) % N_DEV
                rows = pl.ds(c_recv * chunk, chunk)
                for s_i in range(S):
                    r = rs_copy(h, d, s_i)
                    r.wait_recv()
                    r.wait_send()
                    cols = pl.ds(d * half + s_i * sub, sub)
                    out_ref[rows, cols] = out_ref[rows, cols] + comm_ref[h, d, s_i]
                    if h + 1 < N_DEV - 1:
                        rs_copy(h + 1, d, s_i).start()

        c_gelu = 0.7978845608028654
        for d in range(2):
            c_own = (p + 1) % N_DEV if d == 0 else (p - 1) % N_DEV
            rows = pl.ds(c_own * chunk, chunk)
            cols = pl.ds(d * half, half)
            y = out_ref[rows, cols]
            out_ref[rows, cols] = 0.5 * y * (
                1.0 + jnp.tanh(c_gelu * (y + 0.044715 * y * y * y)))

        def ag_copy(t, d, s_i):
            c_send = (p + 1 - t) % N_DEV if d == 0 else (p - 1 + t) % N_DEV
            tgt = right if d == 0 else left
            ref = out_ref.at[pl.ds(c_send * chunk, chunk),
                             pl.ds(d * half + s_i * sub, sub)]
            return pltpu.make_async_remote_copy(
                src_ref=ref,
                dst_ref=ref,
                send_sem=ag_send.at[t, d, s_i],
                recv_sem=ag_recv.at[t, d, s_i],
                device_id=(tgt,),
                device_id_type=pl.DeviceIdType.MESH,
            )

        for d in range(2):
            for s_i in range(S):
                ag_copy(0, d, s_i).start()
        for t in range(N_DEV - 1):
            for d in range(2):
                for s_i in range(S):
                    r = ag_copy(t, d, s_i)
                    r.wait_recv()
                    r.wait_send()
                    if t + 1 < N_DEV - 1:
                        ag_copy(t + 1, d, s_i).start()

    return pl.pallas_call(
        body,
        out_shape=jax.ShapeDtypeStruct((m, n), jnp.float32),
        in_specs=[
            pl.BlockSpec(memory_space=pltpu.VMEM),
            pl.BlockSpec(memory_space=pltpu.VMEM),
        ],
        out_specs=pl.BlockSpec(memory_space=pltpu.VMEM),
        scratch_shapes=[
            pltpu.VMEM((N_DEV - 1, 2, S, chunk, sub), jnp.float32),
            pltpu.SemaphoreType.DMA((N_DEV - 1, 2, S)),
            pltpu.SemaphoreType.DMA((N_DEV - 1, 2, S)),
            pltpu.SemaphoreType.DMA((N_DEV - 1, 2, S)),
            pltpu.SemaphoreType.DMA((N_DEV - 1, 2, S)),
        ],
        compiler_params=pltpu.CompilerParams(collective_id=0),
    )(x, w_mat)


# device time: 193161 ns/iter; 2.6217x vs baseline; 1.0508x over previous
import jax
import jax.numpy as jnp
from jax import lax
from jax.experimental import pallas as pl
from jax.experimental.pallas import tpu as pltpu

N_DEV = 16
S = 4


def kernel(x, w_mat):
    m, k_sh = x.shape
    _, n = w_mat.shape
    chunk = m // N_DEV
    half = n // 2
    sub = half // S

    def body(x_ref, w_ref, out_ref, comm_ref,
             rs_send, rs_recv, ag_send, ag_recv):
        p = lax.axis_index("i")
        left = (p - 1) % N_DEV
        right = (p + 1) % N_DEV

        barrier_sem = pltpu.get_barrier_semaphore()
        for nbr in (left, right):
            pl.semaphore_signal(barrier_sem, inc=1, device_id=(nbr,),
                                device_id_type=pl.DeviceIdType.MESH)

        def gemm_chunk(c):
            rows = pl.ds(c * chunk, chunk)
            out_ref[rows, :] = jnp.dot(x_ref[rows, :], w_ref[:, :],
                                       preferred_element_type=jnp.float32)

        gemm_chunk(p)
        pl.semaphore_wait(barrier_sem, 2)

        def rs_copy(h, d, s_i):
            c_send = (p - h) % N_DEV if d == 0 else (p + h) % N_DEV
            tgt = right if d == 0 else left
            co = d * half + s_i * sub
            return pltpu.make_async_remote_copy(
                src_ref=out_ref.at[pl.ds(c_send * chunk, chunk),
                                   pl.ds(co, sub)],
                dst_ref=comm_ref.at[h, d, s_i],
                send_sem=rs_send.at[h, d, s_i],
                recv_sem=rs_recv.at[h, d, s_i],
                device_id=(tgt,),
                device_id_type=pl.DeviceIdType.MESH,
            )

        for d in range(2):
            for s_i in range(S):
                rs_copy(0, d, s_i).start()
        for j in range(1, N_DEV // 2 + 1):
            gemm_chunk((p - j) % N_DEV)
            if j < N_DEV // 2:
                gemm_chunk((p + j) % N_DEV)
        for h in range(N_DEV - 1):
            for s_i in range(S):
                for d in range(2):
                    c_recv = (p - h - 1) % N_DEV if d == 0 else (p + h + 1) % N_DEV
                    rows = pl.ds(c_recv * chunk, chunk)
                    r = rs_copy(h, d, s_i)
                    r.wait_recv()
                    cols = pl.ds(d * half + s_i * sub, sub)
                    out_ref[rows, cols] = out_ref[rows, cols] + comm_ref[h, d, s_i]
                    if h + 1 < N_DEV - 1:
                        rs_copy(h + 1, d, s_i).start()
                    r.wait_send()

        c_gelu = 0.7978845608028654
        for d in range(2):
            c_own = (p + 1) % N_DEV if d == 0 else (p - 1) % N_DEV
            rows = pl.ds(c_own * chunk, chunk)
            cols = pl.ds(d * half, half)
            y = out_ref[rows, cols]
            out_ref[rows, cols] = 0.5 * y * (
                1.0 + jnp.tanh(c_gelu * (y + 0.044715 * y * y * y)))

        def ag_copy(t, d, s_i):
            c_send = (p + 1 - t) % N_DEV if d == 0 else (p - 1 + t) % N_DEV
            tgt = right if d == 0 else left
            ref = out_ref.at[pl.ds(c_send * chunk, chunk),
                             pl.ds(d * half + s_i * sub, sub)]
            return pltpu.make_async_remote_copy(
                src_ref=ref,
                dst_ref=ref,
                send_sem=ag_send.at[t, d, s_i],
                recv_sem=ag_recv.at[t, d, s_i],
                device_id=(tgt,),
                device_id_type=pl.DeviceIdType.MESH,
            )

        for d in range(2):
            for s_i in range(S):
                ag_copy(0, d, s_i).start()
        for t in range(N_DEV - 1):
            for s_i in range(S):
                for d in range(2):
                    r = ag_copy(t, d, s_i)
                    r.wait_recv()
                    if t + 1 < N_DEV - 1:
                        ag_copy(t + 1, d, s_i).start()
                    r.wait_send()

    return pl.pallas_call(
        body,
        out_shape=jax.ShapeDtypeStruct((m, n), jnp.float32),
        in_specs=[
            pl.BlockSpec(memory_space=pltpu.VMEM),
            pl.BlockSpec(memory_space=pltpu.VMEM),
        ],
        out_specs=pl.BlockSpec(memory_space=pltpu.VMEM),
        scratch_shapes=[
            pltpu.VMEM((N_DEV - 1, 2, S, chunk, sub), jnp.float32),
            pltpu.SemaphoreType.DMA((N_DEV - 1, 2, S)),
            pltpu.SemaphoreType.DMA((N_DEV - 1, 2, S)),
            pltpu.SemaphoreType.DMA((N_DEV - 1, 2, S)),
            pltpu.SemaphoreType.DMA((N_DEV - 1, 2, S)),
        ],
        compiler_params=pltpu.CompilerParams(collective_id=0),
    )(x, w_mat)


# device time: 187812 ns/iter; 2.6963x vs baseline; 1.0285x over previous
import jax
import jax.numpy as jnp
from jax import lax
from jax.experimental import pallas as pl
from jax.experimental.pallas import tpu as pltpu

N_DEV = 16
S = 4


def kernel(x, w_mat):
    m, k_sh = x.shape
    _, n = w_mat.shape
    chunk = m // N_DEV
    half = n // 2
    sub = half // S

    def body(x_ref, w_ref, out_ref, comm_ref,
             rs_send, rs_recv, ag_send, ag_recv):
        p = lax.axis_index("i")
        left = (p - 1) % N_DEV
        right = (p + 1) % N_DEV

        barrier_sem = pltpu.get_barrier_semaphore()
        for nbr in (left, right):
            pl.semaphore_signal(barrier_sem, inc=1, device_id=(nbr,),
                                device_id_type=pl.DeviceIdType.MESH)

        def gemm_chunk(c):
            rows = pl.ds(c * chunk, chunk)
            out_ref[rows, :] = jnp.dot(x_ref[rows, :], w_ref[:, :],
                                       preferred_element_type=jnp.float32)

        gemm_chunk(p)
        pl.semaphore_wait(barrier_sem, 2)

        def rs_copy(h, d, s_i):
            c_send = (p - h) % N_DEV if d == 0 else (p + h) % N_DEV
            tgt = right if d == 0 else left
            co = d * half + s_i * sub
            return pltpu.make_async_remote_copy(
                src_ref=out_ref.at[pl.ds(c_send * chunk, chunk),
                                   pl.ds(co, sub)],
                dst_ref=comm_ref.at[h, d, s_i],
                send_sem=rs_send.at[h, d, s_i],
                recv_sem=rs_recv.at[h, d, s_i],
                device_id=(tgt,),
                device_id_type=pl.DeviceIdType.MESH,
            )

        def ag_copy(t, d, s_i):
            c_send = (p + 1 - t) % N_DEV if d == 0 else (p - 1 + t) % N_DEV
            tgt = right if d == 0 else left
            ref = out_ref.at[pl.ds(c_send * chunk, chunk),
                             pl.ds(d * half + s_i * sub, sub)]
            return pltpu.make_async_remote_copy(
                src_ref=ref,
                dst_ref=ref,
                send_sem=ag_send.at[t, d, s_i],
                recv_sem=ag_recv.at[t, d, s_i],
                device_id=(tgt,),
                device_id_type=pl.DeviceIdType.MESH,
            )

        for d in range(2):
            for s_i in range(S):
                rs_copy(0, d, s_i).start()
        for j in range(1, N_DEV // 2 + 1):
            gemm_chunk((p - j) % N_DEV)
            if j < N_DEV // 2:
                gemm_chunk((p + j) % N_DEV)
        c_gelu = 0.7978845608028654
        for h in range(N_DEV - 1):
            for s_i in range(S):
                for d in range(2):
                    c_recv = (p - h - 1) % N_DEV if d == 0 else (p + h + 1) % N_DEV
                    rows = pl.ds(c_recv * chunk, chunk)
                    r = rs_copy(h, d, s_i)
                    r.wait_recv()
                    cols = pl.ds(d * half + s_i * sub, sub)
                    acc = out_ref[rows, cols] + comm_ref[h, d, s_i]
                    if h + 1 < N_DEV - 1:
                        out_ref[rows, cols] = acc
                        rs_copy(h + 1, d, s_i).start()
                    else:
                        out_ref[rows, cols] = 0.5 * acc * (
                            1.0 + jnp.tanh(c_gelu * (acc + 0.044715 * acc * acc * acc)))
                        ag_copy(0, d, s_i).start()
                    r.wait_send()

        for t in range(N_DEV - 1):
            for s_i in range(S):
                for d in range(2):
                    r = ag_copy(t, d, s_i)
                    r.wait_recv()
                    if t + 1 < N_DEV - 1:
                        ag_copy(t + 1, d, s_i).start()
                    r.wait_send()

    return pl.pallas_call(
        body,
        out_shape=jax.ShapeDtypeStruct((m, n), jnp.float32),
        in_specs=[
            pl.BlockSpec(memory_space=pltpu.VMEM),
            pl.BlockSpec(memory_space=pltpu.VMEM),
        ],
        out_specs=pl.BlockSpec(memory_space=pltpu.VMEM),
        scratch_shapes=[
            pltpu.VMEM((N_DEV - 1, 2, S, chunk, sub), jnp.float32),
            pltpu.SemaphoreType.DMA((N_DEV - 1, 2, S)),
            pltpu.SemaphoreType.DMA((N_DEV - 1, 2, S)),
            pltpu.SemaphoreType.DMA((N_DEV - 1, 2, S)),
            pltpu.SemaphoreType.DMA((N_DEV - 1, 2, S)),
        ],
        compiler_params=pltpu.CompilerParams(collective_id=0),
    )(x, w_mat)
